# Initial kernel scaffold; baseline (speedup 1.0000x reference)
import jax
import jax.numpy as jnp
from jax.experimental import pallas as pl


def kernel(pts, W_m1_1, b_m1_1, W_m1_2, b_m1_2, W_m1_3, b_m1_3, W_g1_1, b_g1_1, W_g1_2, b_g1_2, W_g2_1, b_g2_1, W_g2_2, b_g2_2, W_m2_1, b_m2_1, W_m2_2, b_m2_2):
    raise NotImplementedError("write your pallas kernel here")


# speedup vs baseline: 6.8509x; 6.8509x over previous
import functools

import jax
import jax.numpy as jnp
from jax.experimental import pallas as pl
from jax.experimental.pallas import tpu as pltpu

_INTERPRET = False

B, N, KNN = 8, 2048, 16
R = 256
NB = N // R
NEG_INF = float("-inf")
INT_MIN = -(2 ** 31)


def _relu(x):
    return jnp.maximum(x, 0.0)


def _dot_t(a, b):
    return jax.lax.dot_general(a, b, (((1,), (1,)), ((), ())),
                               precision=jax.lax.Precision.HIGHEST,
                               preferred_element_type=jnp.float32)


def _dot_t_bf(a, b):
    return jax.lax.dot_general(
        a.astype(jnp.bfloat16), b.astype(jnp.bfloat16),
        (((1,), (1,)), ((), ())), preferred_element_type=jnp.float32)


def _pd_tile(rows, allpts):
    inner = _dot_t_bf(rows, allpts)
    onesc = jnp.ones((1, allpts.shape[1]), jnp.float32)
    xx_all = _dot_t(onesc, allpts * allpts)
    xx_rows = jnp.sum(rows * rows, axis=1, keepdims=True)
    return (2.0 * inner - xx_rows) - xx_all


def _sortable_key(pd):
    bits = jax.lax.bitcast_convert_type(pd, jnp.int32)
    key = jnp.where(bits < 0, bits ^ jnp.int32(0x7FFFFFFF), bits)
    return key & jnp.int32(-2048)


def _stage1_kernel(pts_ref, w1_ref, b1_ref, w2_ref, b2_ref, w3_ref, b3_ref,
                   h_ref):
    i = pl.program_id(1)
    x_all = pts_ref[0]
    rows = pts_ref[0, pl.ds(i * R, R), :]
    pd = _pd_tile(rows, x_all)

    lane = jax.lax.broadcasted_iota(jnp.int32, (R, N), 1)
    nbrs = []
    for _ in range(2):
        m = jnp.max(pd, axis=1, keepdims=True)
        eq = pd == m
        amin = jnp.min(jnp.where(eq, lane, N), axis=1, keepdims=True)
        onehot = lane == amin
        nbrs.append(jax.lax.dot_general(
            onehot.astype(jnp.float32), x_all, (((1,), (0,)), ((), ())),
            precision=jax.lax.Precision.HIGHEST,
            preferred_element_type=jnp.float32))
        pd = jnp.where(onehot, NEG_INF, pd)
    nb0, nb1 = nbrs

    feats = jnp.concatenate(
        [rows, nb0[:, 0:1] * nb1, nb0[:, 1:2] * nb1, nb0[:, 2:3] * nb1],
        axis=1)
    h = _relu(_dot_t_bf(feats, w1_ref[...]) + b1_ref[...])
    h = _relu(_dot_t_bf(h, w2_ref[...]) + b2_ref[...])
    h = _relu(_dot_t_bf(h, w3_ref[...]) + b3_ref[...])
    h_ref[0] = h


def _knn_maxpool(feats_all, rows):
    pd = _pd_tile(rows, feats_all)
    lane = jax.lax.broadcasted_iota(jnp.int32, (R, N), 1)
    kp = _sortable_key(pd) | (jnp.int32(2047) - lane)
    pooled = jnp.full((R, feats_all.shape[1]), NEG_INF, jnp.float32)
    for _ in range(KNN):
        m = jnp.max(kp, axis=1, keepdims=True)
        sel = kp == m
        g = jax.lax.dot_general(
            sel.astype(jnp.float32), feats_all, (((1,), (0,)), ((), ())),
            precision=jax.lax.Precision.HIGHEST,
            preferred_element_type=jnp.float32)
        pooled = jnp.maximum(pooled, g)
        kp = jnp.where(sel, INT_MIN, kp)
    return pooled


def _stage2_kernel(h_ref, wg1_ref, bg1_ref, wg2_ref, bg2_ref, g_ref):
    i = pl.program_id(1)
    h_all = h_ref[0]
    rows = h_ref[0, pl.ds(i * R, R), :]
    pooled = _knn_maxpool(h_all, rows)
    g = _relu(_dot_t_bf(pooled, wg1_ref[...]) + bg1_ref[...])
    g_ref[0] = _dot_t_bf(g, wg2_ref[...]) + bg2_ref[...]


def _stage3_kernel(g_ref, wg1_ref, bg1_ref, wg2_ref, bg2_ref, glob_ref):
    i = pl.program_id(1)
    g_all = g_ref[0]
    rows = g_ref[0, pl.ds(i * R, R), :]
    pooled = _knn_maxpool(g_all, rows)
    t = _relu(_dot_t_bf(pooled, wg1_ref[...]) + bg1_ref[...])
    t = _dot_t_bf(t, wg2_ref[...]) + bg2_ref[...]
    part = jnp.max(t, axis=0, keepdims=True)[None]

    @pl.when(i == 0)
    def _():
        glob_ref[...] = part

    @pl.when(i != 0)
    def _():
        glob_ref[...] = jnp.maximum(glob_ref[...], part)


def _stage4_kernel(glob_ref, w1_ref, b1_ref, w2_ref, b2_ref, out_ref):
    c = _relu(_dot_t_bf(glob_ref[...], w1_ref[...]) + b1_ref[...])
    out_ref[...] = _dot_t_bf(c, w2_ref[...]) + b2_ref[...]


def _full(shape):
    return pl.BlockSpec(shape, lambda *_: tuple(0 for _ in shape))


def _batch_full(shape):
    return pl.BlockSpec((1,) + shape, lambda b, i: (b, 0, 0))


@jax.jit
def kernel(pts, W_m1_1, b_m1_1, W_m1_2, b_m1_2, W_m1_3, b_m1_3,
           W_g1_1, b_g1_1, W_g1_2, b_g1_2, W_g2_1, b_g2_1, W_g2_2, b_g2_2,
           W_m2_1, b_m2_1, W_m2_2, b_m2_2):
    row2 = lambda v: v.reshape(1, -1)
    call = functools.partial(
        pl.pallas_call, grid=(B, NB), interpret=_INTERPRET,
        compiler_params=pltpu.CompilerParams(
            dimension_semantics=("parallel", "arbitrary")))

    h = call(
        _stage1_kernel,
        in_specs=[_batch_full((N, 3)), _full((64, 12)), _full((1, 64)),
                  _full((64, 64)), _full((1, 64)), _full((64, 64)),
                  _full((1, 64))],
        out_specs=pl.BlockSpec((1, R, 64), lambda b, i: (b, i, 0)),
        out_shape=jax.ShapeDtypeStruct((B, N, 64), jnp.float32),
    )(pts, W_m1_1, row2(b_m1_1), W_m1_2, row2(b_m1_2), W_m1_3, row2(b_m1_3))

    g = call(
        _stage2_kernel,
        in_specs=[_batch_full((N, 64)), _full((64, 64)), _full((1, 64)),
                  _full((128, 64)), _full((1, 128))],
        out_specs=pl.BlockSpec((1, R, 128), lambda b, i: (b, i, 0)),
        out_shape=jax.ShapeDtypeStruct((B, N, 128), jnp.float32),
    )(h, W_g1_1, row2(b_g1_1), W_g1_2, row2(b_g1_2))

    glob = call(
        _stage3_kernel,
        in_specs=[_batch_full((N, 128)), _full((128, 128)), _full((1, 128)),
                  _full((1024, 128)), _full((1, 1024))],
        out_specs=pl.BlockSpec((1, 1, 1024), lambda b, i: (b, 0, 0)),
        out_shape=jax.ShapeDtypeStruct((B, 1, 1024), jnp.float32),
    )(g, W_g2_1, row2(b_g2_1), W_g2_2, row2(b_g2_2))

    c = pl.pallas_call(
        _stage4_kernel, grid=(1,), interpret=_INTERPRET,
        in_specs=[pl.BlockSpec((B, 1024), lambda _: (0, 0)),
                  pl.BlockSpec((512, 1024), lambda _: (0, 0)),
                  pl.BlockSpec((1, 512), lambda _: (0, 0)),
                  pl.BlockSpec((512, 512), lambda _: (0, 0)),
                  pl.BlockSpec((1, 512), lambda _: (0, 0))],
        out_specs=pl.BlockSpec((B, 512), lambda _: (0, 0)),
        out_shape=jax.ShapeDtypeStruct((B, 512), jnp.float32),
    )(glob.reshape(B, 1024), W_m2_1, row2(b_m2_1), W_m2_2, row2(b_m2_2))

    return c[:, :, None]


# speedup vs baseline: 11.3935x; 1.6631x over previous
import functools

import jax
import jax.numpy as jnp
from jax.experimental import pallas as pl
from jax.experimental.pallas import tpu as pltpu

_INTERPRET = False

B, N, KNN = 8, 2048, 16
R = 256
NB = N // R
NEG_INF = float("-inf")
INT_MIN = -(2 ** 31)


def _relu(x):
    return jnp.maximum(x, 0.0)


def _dot_t(a, b):
    return jax.lax.dot_general(a, b, (((1,), (1,)), ((), ())),
                               precision=jax.lax.Precision.HIGHEST,
                               preferred_element_type=jnp.float32)


def _dot_t_bf(a, b):
    return jax.lax.dot_general(
        a.astype(jnp.bfloat16), b.astype(jnp.bfloat16),
        (((1,), (1,)), ((), ())), preferred_element_type=jnp.float32)


def _pd_tile(rows, allpts):
    inner = _dot_t_bf(rows, allpts)
    onesc = jnp.ones((1, allpts.shape[1]), jnp.float32)
    xx_all = _dot_t(onesc, allpts * allpts)
    xx_rows = jnp.sum(rows * rows, axis=1, keepdims=True)
    return (2.0 * inner - xx_rows) - xx_all


def _split3(x):
    p1 = x.astype(jnp.bfloat16)
    r1 = x - p1.astype(jnp.float32)
    p2 = r1.astype(jnp.bfloat16)
    r2 = r1 - p2.astype(jnp.float32)
    return p1, p2, r2.astype(jnp.bfloat16)


def _gather3(sel_bf, parts):
    acc = None
    for p in parts:
        t = jax.lax.dot_general(sel_bf, p, (((1,), (0,)), ((), ())),
                                preferred_element_type=jnp.float32)
        acc = t if acc is None else acc + t
    return acc


def _sortable_key(pd):
    bits = jax.lax.bitcast_convert_type(pd, jnp.int32)
    key = jnp.where(bits < 0, bits ^ jnp.int32(0x7FFFFFFF), bits)
    return key & jnp.int32(-2048)


def _stage1_kernel(pts_ref, w1_ref, b1_ref, w2_ref, b2_ref, w3_ref, b3_ref,
                   h_ref):
    i = pl.program_id(1)
    x_all = pts_ref[0]
    rows = pts_ref[0, pl.ds(i * R, R), :]
    pd = _pd_tile(rows, x_all)

    lane = jax.lax.broadcasted_iota(jnp.int32, (R, N), 1)
    parts = _split3(x_all)
    nbrs = []
    for _ in range(2):
        m = jnp.max(pd, axis=1, keepdims=True)
        eq = pd == m
        amin = jnp.min(jnp.where(eq, lane, N), axis=1, keepdims=True)
        onehot = lane == amin
        nbrs.append(_gather3(onehot.astype(jnp.bfloat16), parts))
        pd = jnp.where(onehot, NEG_INF, pd)
    nb0, nb1 = nbrs

    feats = jnp.concatenate(
        [rows, nb0[:, 0:1] * nb1, nb0[:, 1:2] * nb1, nb0[:, 2:3] * nb1],
        axis=1)
    h = _relu(_dot_t_bf(feats, w1_ref[...]) + b1_ref[...])
    h = _relu(_dot_t_bf(h, w2_ref[...]) + b2_ref[...])
    h = _relu(_dot_t_bf(h, w3_ref[...]) + b3_ref[...])
    h_ref[0] = h


def _knn_maxpool(feats_all, rows):
    pd = _pd_tile(rows, feats_all)
    parts = _split3(feats_all)
    lane = jax.lax.broadcasted_iota(jnp.int32, (R, N), 1)
    kp = _sortable_key(pd) | (jnp.int32(2047) - lane)
    pooled = jnp.full((R, feats_all.shape[1]), NEG_INF, jnp.float32)
    for _ in range(KNN):
        m = jnp.max(kp, axis=1, keepdims=True)
        sel = kp == m
        g = _gather3(sel.astype(jnp.bfloat16), parts)
        pooled = jnp.maximum(pooled, g)
        kp = jnp.where(sel, INT_MIN, kp)
    return pooled


def _stage2_kernel(h_ref, wg1_ref, bg1_ref, wg2_ref, bg2_ref, g_ref):
    i = pl.program_id(1)
    h_all = h_ref[0]
    rows = h_ref[0, pl.ds(i * R, R), :]
    pooled = _knn_maxpool(h_all, rows)
    g = _relu(_dot_t_bf(pooled, wg1_ref[...]) + bg1_ref[...])
    g_ref[0] = _dot_t_bf(g, wg2_ref[...]) + bg2_ref[...]


def _stage3_kernel(g_ref, wg1_ref, bg1_ref, wg2_ref, bg2_ref, glob_ref):
    i = pl.program_id(1)
    g_all = g_ref[0]
    rows = g_ref[0, pl.ds(i * R, R), :]
    pooled = _knn_maxpool(g_all, rows)
    t = _relu(_dot_t_bf(pooled, wg1_ref[...]) + bg1_ref[...])
    t = _dot_t_bf(t, wg2_ref[...]) + bg2_ref[...]
    part = jnp.max(t, axis=0, keepdims=True)[None]

    @pl.when(i == 0)
    def _():
        glob_ref[...] = part

    @pl.when(i != 0)
    def _():
        glob_ref[...] = jnp.maximum(glob_ref[...], part)


def _stage4_kernel(glob_ref, w1_ref, b1_ref, w2_ref, b2_ref, out_ref):
    c = _relu(_dot_t_bf(glob_ref[...], w1_ref[...]) + b1_ref[...])
    out_ref[...] = _dot_t_bf(c, w2_ref[...]) + b2_ref[...]


def _full(shape):
    return pl.BlockSpec(shape, lambda *_: tuple(0 for _ in shape))


def _batch_full(shape):
    return pl.BlockSpec((1,) + shape, lambda b, i: (b, 0, 0))


@jax.jit
def kernel(pts, W_m1_1, b_m1_1, W_m1_2, b_m1_2, W_m1_3, b_m1_3,
           W_g1_1, b_g1_1, W_g1_2, b_g1_2, W_g2_1, b_g2_1, W_g2_2, b_g2_2,
           W_m2_1, b_m2_1, W_m2_2, b_m2_2):
    row2 = lambda v: v.reshape(1, -1)
    call = functools.partial(
        pl.pallas_call, grid=(B, NB), interpret=_INTERPRET,
        compiler_params=pltpu.CompilerParams(
            dimension_semantics=("parallel", "arbitrary")))

    h = call(
        _stage1_kernel,
        in_specs=[_batch_full((N, 3)), _full((64, 12)), _full((1, 64)),
                  _full((64, 64)), _full((1, 64)), _full((64, 64)),
                  _full((1, 64))],
        out_specs=pl.BlockSpec((1, R, 64), lambda b, i: (b, i, 0)),
        out_shape=jax.ShapeDtypeStruct((B, N, 64), jnp.float32),
    )(pts, W_m1_1, row2(b_m1_1), W_m1_2, row2(b_m1_2), W_m1_3, row2(b_m1_3))

    g = call(
        _stage2_kernel,
        in_specs=[_batch_full((N, 64)), _full((64, 64)), _full((1, 64)),
                  _full((128, 64)), _full((1, 128))],
        out_specs=pl.BlockSpec((1, R, 128), lambda b, i: (b, i, 0)),
        out_shape=jax.ShapeDtypeStruct((B, N, 128), jnp.float32),
    )(h, W_g1_1, row2(b_g1_1), W_g1_2, row2(b_g1_2))

    glob = call(
        _stage3_kernel,
        in_specs=[_batch_full((N, 128)), _full((128, 128)), _full((1, 128)),
                  _full((1024, 128)), _full((1, 1024))],
        out_specs=pl.BlockSpec((1, 1, 1024), lambda b, i: (b, 0, 0)),
        out_shape=jax.ShapeDtypeStruct((B, 1, 1024), jnp.float32),
    )(g, W_g2_1, row2(b_g2_1), W_g2_2, row2(b_g2_2))

    c = pl.pallas_call(
        _stage4_kernel, grid=(1,), interpret=_INTERPRET,
        in_specs=[pl.BlockSpec((B, 1024), lambda _: (0, 0)),
                  pl.BlockSpec((512, 1024), lambda _: (0, 0)),
                  pl.BlockSpec((1, 512), lambda _: (0, 0)),
                  pl.BlockSpec((512, 512), lambda _: (0, 0)),
                  pl.BlockSpec((1, 512), lambda _: (0, 0))],
        out_specs=pl.BlockSpec((B, 512), lambda _: (0, 0)),
        out_shape=jax.ShapeDtypeStruct((B, 512), jnp.float32),
    )(glob.reshape(B, 1024), W_m2_1, row2(b_m2_1), W_m2_2, row2(b_m2_2))

    return c[:, :, None]


# speedup vs baseline: 16.0215x; 1.4062x over previous
import functools
import inspect

import jax
import jax.numpy as jnp
import numpy as np
from jax.experimental import pallas as pl
from jax.experimental.pallas import tpu as pltpu

_INTERPRET = False

B, N, KNN = 8, 2048, 16
R = 256
NB = N // R
NEG_INF = float("-inf")
INT_MIN = -(2 ** 31)


def _relu(x):
    return jnp.maximum(x, 0.0)


def _dot_t(a, b):
    return jax.lax.dot_general(a, b, (((1,), (1,)), ((), ())),
                               precision=jax.lax.Precision.HIGHEST,
                               preferred_element_type=jnp.float32)


def _dot_t_bf(a, b):
    return jax.lax.dot_general(
        a.astype(jnp.bfloat16), b.astype(jnp.bfloat16),
        (((1,), (1,)), ((), ())), preferred_element_type=jnp.float32)


def _pd_tile(rows, allpts):
    inner = _dot_t_bf(rows, allpts)
    onesc = jnp.ones((1, allpts.shape[1]), jnp.float32)
    xx_all = _dot_t(onesc, allpts * allpts)
    xx_rows = jnp.sum(rows * rows, axis=1, keepdims=True)
    return (2.0 * inner - xx_rows) - xx_all


def _split3(x):
    p1 = x.astype(jnp.bfloat16)
    r1 = x - p1.astype(jnp.float32)
    p2 = r1.astype(jnp.bfloat16)
    r2 = r1 - p2.astype(jnp.float32)
    return p1, p2, r2.astype(jnp.bfloat16)


def _gather3(sel_bf, parts):
    acc = None
    for p in parts:
        t = jax.lax.dot_general(sel_bf, p, (((1,), (0,)), ((), ())),
                                preferred_element_type=jnp.float32)
        acc = t if acc is None else acc + t
    return acc


def _sortable_key(pd):
    bits = jax.lax.bitcast_convert_type(pd, jnp.int32)
    key = jnp.where(bits < 0, bits ^ jnp.int32(0x7FFFFFFF), bits)
    return key & jnp.int32(-2048)


def _stage1_kernel(pts_ref, w1_ref, b1_ref, w2_ref, b2_ref, w3_ref, b3_ref,
                   h_ref):
    i = pl.program_id(1)
    x_all = pts_ref[0]
    rows = pts_ref[0, pl.ds(i * R, R), :]
    pd = _pd_tile(rows, x_all)

    lane = jax.lax.broadcasted_iota(jnp.int32, (R, N), 1)
    parts = _split3(x_all)
    nbrs = []
    for _ in range(2):
        m = jnp.max(pd, axis=1, keepdims=True)
        eq = pd == m
        amin = jnp.min(jnp.where(eq, lane, N), axis=1, keepdims=True)
        onehot = lane == amin
        nbrs.append(_gather3(onehot.astype(jnp.bfloat16), parts))
        pd = jnp.where(onehot, NEG_INF, pd)
    nb0, nb1 = nbrs

    feats = jnp.concatenate(
        [rows, nb0[:, 0:1] * nb1, nb0[:, 1:2] * nb1, nb0[:, 2:3] * nb1],
        axis=1)
    h = _relu(_dot_t_bf(feats, w1_ref[...]) + b1_ref[...])
    h = _relu(_dot_t_bf(h, w2_ref[...]) + b2_ref[...])
    h = _relu(_dot_t_bf(h, w3_ref[...]) + b3_ref[...])
    h_ref[0] = h


def _knn_maxpool(feats_all, rows):
    pd = _pd_tile(rows, feats_all)
    parts = _split3(feats_all)
    lane = jax.lax.broadcasted_iota(jnp.int32, (R, N), 1)
    kp = _sortable_key(pd) | (jnp.int32(2047) - lane)
    pooled = jnp.full((R, feats_all.shape[1]), NEG_INF, jnp.float32)
    for _ in range(KNN):
        m = jnp.max(kp, axis=1, keepdims=True)
        sel = kp == m
        g = _gather3(sel.astype(jnp.bfloat16), parts)
        pooled = jnp.maximum(pooled, g)
        kp = jnp.where(sel, INT_MIN, kp)
    return pooled


def _stage2_kernel(h_ref, wg1_ref, bg1_ref, wg2_ref, bg2_ref, g_ref):
    i = pl.program_id(1)
    h_all = h_ref[0]
    rows = h_ref[0, pl.ds(i * R, R), :]
    pooled = _knn_maxpool(h_all, rows)
    g = _relu(_dot_t_bf(pooled, wg1_ref[...]) + bg1_ref[...])
    g_ref[0] = _dot_t_bf(g, wg2_ref[...]) + bg2_ref[...]


def _stage3_kernel(g_ref, wg1_ref, bg1_ref, wg2_ref, bg2_ref, glob_ref):
    i = pl.program_id(1)
    g_all = g_ref[0]
    rows = g_ref[0, pl.ds(i * R, R), :]
    pooled = _knn_maxpool(g_all, rows)
    t = _relu(_dot_t_bf(pooled, wg1_ref[...]) + bg1_ref[...])
    t = _dot_t_bf(t, wg2_ref[...]) + bg2_ref[...]
    part = jnp.max(t, axis=0, keepdims=True)[None]

    @pl.when(i == 0)
    def _():
        glob_ref[...] = part

    @pl.when(i != 0)
    def _():
        glob_ref[...] = jnp.maximum(glob_ref[...], part)


def _stage4_kernel(glob_ref, w1_ref, b1_ref, w2_ref, b2_ref, out_ref):
    c = _relu(_dot_t_bf(glob_ref[...], w1_ref[...]) + b1_ref[...])
    out_ref[...] = _dot_t_bf(c, w2_ref[...]) + b2_ref[...]


def _full(shape):
    return pl.BlockSpec(shape, lambda *_: tuple(0 for _ in shape))


def _batch_full(shape):
    return pl.BlockSpec((1,) + shape, lambda b, i: (b, 0, 0))


def _pipeline(pts, W_m1_1, b_m1_1, W_m1_2, b_m1_2, W_m1_3, b_m1_3,
              W_g1_1, b_g1_1, W_g1_2, b_g1_2, W_g2_1, b_g2_1, W_g2_2, b_g2_2,
              W_m2_1, b_m2_1, W_m2_2, b_m2_2):
    B = pts.shape[0]
    row2 = lambda v: v.reshape(1, -1)
    call = functools.partial(
        pl.pallas_call, grid=(B, NB), interpret=_INTERPRET,
        compiler_params=pltpu.CompilerParams(
            dimension_semantics=("parallel", "arbitrary")))

    h = call(
        _stage1_kernel,
        in_specs=[_batch_full((N, 3)), _full((64, 12)), _full((1, 64)),
                  _full((64, 64)), _full((1, 64)), _full((64, 64)),
                  _full((1, 64))],
        out_specs=pl.BlockSpec((1, R, 64), lambda b, i: (b, i, 0)),
        out_shape=jax.ShapeDtypeStruct((B, N, 64), jnp.float32),
    )(pts, W_m1_1, row2(b_m1_1), W_m1_2, row2(b_m1_2), W_m1_3, row2(b_m1_3))

    g = call(
        _stage2_kernel,
        in_specs=[_batch_full((N, 64)), _full((64, 64)), _full((1, 64)),
                  _full((128, 64)), _full((1, 128))],
        out_specs=pl.BlockSpec((1, R, 128), lambda b, i: (b, i, 0)),
        out_shape=jax.ShapeDtypeStruct((B, N, 128), jnp.float32),
    )(h, W_g1_1, row2(b_g1_1), W_g1_2, row2(b_g1_2))

    glob = call(
        _stage3_kernel,
        in_specs=[_batch_full((N, 128)), _full((128, 128)), _full((1, 128)),
                  _full((1024, 128)), _full((1, 1024))],
        out_specs=pl.BlockSpec((1, 1, 1024), lambda b, i: (b, 0, 0)),
        out_shape=jax.ShapeDtypeStruct((B, 1, 1024), jnp.float32),
    )(g, W_g2_1, row2(b_g2_1), W_g2_2, row2(b_g2_2))

    c = pl.pallas_call(
        _stage4_kernel, grid=(1,), interpret=_INTERPRET,
        in_specs=[pl.BlockSpec((B, 1024), lambda _: (0, 0)),
                  pl.BlockSpec((512, 1024), lambda _: (0, 0)),
                  pl.BlockSpec((1, 512), lambda _: (0, 0)),
                  pl.BlockSpec((512, 512), lambda _: (0, 0)),
                  pl.BlockSpec((1, 512), lambda _: (0, 0))],
        out_specs=pl.BlockSpec((B, 512), lambda _: (0, 0)),
        out_shape=jax.ShapeDtypeStruct((B, 512), jnp.float32),
    )(glob.reshape(B, 1024), W_m2_1, row2(b_m2_1), W_m2_2, row2(b_m2_2))

    return c[:, :, None]


try:
    from jax import shard_map as _shard_map
except ImportError:
    from jax.experimental.shard_map import shard_map as _shard_map

_SM_KW = {}
_sm_params = inspect.signature(_shard_map).parameters
if "check_rep" in _sm_params:
    _SM_KW["check_rep"] = False
elif "check_vma" in _sm_params:
    _SM_KW["check_vma"] = False


@jax.jit
def kernel(pts, W_m1_1, b_m1_1, W_m1_2, b_m1_2, W_m1_3, b_m1_3,
           W_g1_1, b_g1_1, W_g1_2, b_g1_2, W_g2_1, b_g2_1, W_g2_2, b_g2_2,
           W_m2_1, b_m2_1, W_m2_2, b_m2_2):
    args = (pts, W_m1_1, b_m1_1, W_m1_2, b_m1_2, W_m1_3, b_m1_3,
            W_g1_1, b_g1_1, W_g1_2, b_g1_2, W_g2_1, b_g2_1, W_g2_2, b_g2_2,
            W_m2_1, b_m2_1, W_m2_2, b_m2_2)
    devs = jax.devices()
    if len(devs) >= 2 and pts.shape[0] % 2 == 0:
        mesh = jax.sharding.Mesh(np.asarray(devs[:2]), ("d",))
        P = jax.sharding.PartitionSpec
        in_specs = (P("d"),) + (P(),) * 18
        f = _shard_map(_pipeline, mesh=mesh, in_specs=in_specs,
                       out_specs=P("d"), **_SM_KW)
        return f(*args)
    return _pipeline(*args)


# speedup vs baseline: 18.5648x; 1.1587x over previous
import functools
import inspect

import jax
import jax.numpy as jnp
import numpy as np
from jax.experimental import pallas as pl
from jax.experimental.pallas import tpu as pltpu

_INTERPRET = False

B, N, KNN = 8, 2048, 16
R = 256
NB = N // R
NEG_INF = float("-inf")
INT_MIN = -(2 ** 31)


def _relu(x):
    return jnp.maximum(x, 0.0)


def _dot_t(a, b):
    return jax.lax.dot_general(a, b, (((1,), (1,)), ((), ())),
                               precision=jax.lax.Precision.HIGHEST,
                               preferred_element_type=jnp.float32)


def _dot_t_bf(a, b):
    return jax.lax.dot_general(
        a.astype(jnp.bfloat16), b.astype(jnp.bfloat16),
        (((1,), (1,)), ((), ())), preferred_element_type=jnp.float32)


def _pd_tile(rows, allpts):
    inner = _dot_t_bf(rows, allpts)
    onesc = jnp.ones((1, allpts.shape[1]), jnp.float32)
    xx_all = _dot_t(onesc, allpts * allpts)
    xx_rows = jnp.sum(rows * rows, axis=1, keepdims=True)
    return (2.0 * inner - xx_rows) - xx_all


def _split3(x):
    p1 = x.astype(jnp.bfloat16)
    r1 = x - p1.astype(jnp.float32)
    p2 = r1.astype(jnp.bfloat16)
    r2 = r1 - p2.astype(jnp.float32)
    return p1, p2, r2.astype(jnp.bfloat16)


def _gather3(sel_bf, parts):
    acc = None
    for p in parts:
        t = jax.lax.dot_general(sel_bf, p, (((1,), (0,)), ((), ())),
                                preferred_element_type=jnp.float32)
        acc = t if acc is None else acc + t
    return acc


def _sortable_key(pd):
    bits = jax.lax.bitcast_convert_type(pd, jnp.int32)
    key = jnp.where(bits < 0, bits ^ jnp.int32(0x7FFFFFFF), bits)
    return key & jnp.int32(-2048)


def _stage1_kernel(pts_ref, w1_ref, b1_ref, w2_ref, b2_ref, w3_ref, b3_ref,
                   h_ref):
    i = pl.program_id(1)
    x_all = pts_ref[0]
    rows = pts_ref[0, pl.ds(i * R, R), :]
    pd = _pd_tile(rows, x_all)

    lane = jax.lax.broadcasted_iota(jnp.int32, (R, N), 1)
    parts = _split3(x_all)
    nbrs = []
    for _ in range(2):
        m = jnp.max(pd, axis=1, keepdims=True)
        eq = pd == m
        amin = jnp.min(jnp.where(eq, lane, N), axis=1, keepdims=True)
        onehot = lane == amin
        nbrs.append(_gather3(onehot.astype(jnp.bfloat16), parts))
        pd = jnp.where(onehot, NEG_INF, pd)
    nb0, nb1 = nbrs

    feats = jnp.concatenate(
        [rows, nb0[:, 0:1] * nb1, nb0[:, 1:2] * nb1, nb0[:, 2:3] * nb1],
        axis=1)
    h = _relu(_dot_t_bf(feats, w1_ref[...]) + b1_ref[...])
    h = _relu(_dot_t_bf(h, w2_ref[...]) + b2_ref[...])
    h = _relu(_dot_t_bf(h, w3_ref[...]) + b3_ref[...])
    h_ref[0] = h


def _knn_maxpool(feats_all, rows):
    pd = _pd_tile(rows, feats_all)
    parts = _split3(feats_all)[:2]
    lane = jax.lax.broadcasted_iota(jnp.int32, (R, N), 1)
    kp = _sortable_key(pd) | (jnp.int32(2047) - lane)
    pooled = jnp.full((R, feats_all.shape[1]), NEG_INF, jnp.float32)
    for _ in range(KNN):
        m = jnp.max(kp, axis=1, keepdims=True)
        sel = kp == m
        g = _gather3(sel.astype(jnp.bfloat16), parts)
        pooled = jnp.maximum(pooled, g)
        kp = jnp.where(sel, INT_MIN, kp)
    return pooled


def _stage2_kernel(h_ref, wg1_ref, bg1_ref, wg2_ref, bg2_ref, g_ref):
    i = pl.program_id(1)
    h_all = h_ref[0]
    rows = h_ref[0, pl.ds(i * R, R), :]
    pooled = _knn_maxpool(h_all, rows)
    g = _relu(_dot_t_bf(pooled, wg1_ref[...]) + bg1_ref[...])
    g_ref[0] = _dot_t_bf(g, wg2_ref[...]) + bg2_ref[...]


def _stage3_kernel(g_ref, wg1_ref, bg1_ref, wg2_ref, bg2_ref, glob_ref):
    i = pl.program_id(1)
    g_all = g_ref[0]
    rows = g_ref[0, pl.ds(i * R, R), :]
    pooled = _knn_maxpool(g_all, rows)
    t = _relu(_dot_t_bf(pooled, wg1_ref[...]) + bg1_ref[...])
    t = _dot_t_bf(t, wg2_ref[...]) + bg2_ref[...]
    part = jnp.max(t, axis=0, keepdims=True)[None]

    @pl.when(i == 0)
    def _():
        glob_ref[...] = part

    @pl.when(i != 0)
    def _():
        glob_ref[...] = jnp.maximum(glob_ref[...], part)


def _stage4_kernel(glob_ref, w1_ref, b1_ref, w2_ref, b2_ref, out_ref):
    c = _relu(_dot_t_bf(glob_ref[...], w1_ref[...]) + b1_ref[...])
    out_ref[...] = _dot_t_bf(c, w2_ref[...]) + b2_ref[...]


def _full(shape):
    return pl.BlockSpec(shape, lambda *_: tuple(0 for _ in shape))


def _batch_full(shape):
    return pl.BlockSpec((1,) + shape, lambda b, i: (b, 0, 0))


def _pipeline(pts, W_m1_1, b_m1_1, W_m1_2, b_m1_2, W_m1_3, b_m1_3,
              W_g1_1, b_g1_1, W_g1_2, b_g1_2, W_g2_1, b_g2_1, W_g2_2, b_g2_2,
              W_m2_1, b_m2_1, W_m2_2, b_m2_2):
    B = pts.shape[0]
    row2 = lambda v: v.reshape(1, -1)
    call = functools.partial(
        pl.pallas_call, grid=(B, NB), interpret=_INTERPRET,
        compiler_params=pltpu.CompilerParams(
            dimension_semantics=("parallel", "arbitrary")))

    h = call(
        _stage1_kernel,
        in_specs=[_batch_full((N, 3)), _full((64, 12)), _full((1, 64)),
                  _full((64, 64)), _full((1, 64)), _full((64, 64)),
                  _full((1, 64))],
        out_specs=pl.BlockSpec((1, R, 64), lambda b, i: (b, i, 0)),
        out_shape=jax.ShapeDtypeStruct((B, N, 64), jnp.float32),
    )(pts, W_m1_1, row2(b_m1_1), W_m1_2, row2(b_m1_2), W_m1_3, row2(b_m1_3))

    g = call(
        _stage2_kernel,
        in_specs=[_batch_full((N, 64)), _full((64, 64)), _full((1, 64)),
                  _full((128, 64)), _full((1, 128))],
        out_specs=pl.BlockSpec((1, R, 128), lambda b, i: (b, i, 0)),
        out_shape=jax.ShapeDtypeStruct((B, N, 128), jnp.float32),
    )(h, W_g1_1, row2(b_g1_1), W_g1_2, row2(b_g1_2))

    glob = call(
        _stage3_kernel,
        in_specs=[_batch_full((N, 128)), _full((128, 128)), _full((1, 128)),
                  _full((1024, 128)), _full((1, 1024))],
        out_specs=pl.BlockSpec((1, 1, 1024), lambda b, i: (b, 0, 0)),
        out_shape=jax.ShapeDtypeStruct((B, 1, 1024), jnp.float32),
    )(g, W_g2_1, row2(b_g2_1), W_g2_2, row2(b_g2_2))

    c = pl.pallas_call(
        _stage4_kernel, grid=(1,), interpret=_INTERPRET,
        in_specs=[pl.BlockSpec((B, 1024), lambda _: (0, 0)),
                  pl.BlockSpec((512, 1024), lambda _: (0, 0)),
                  pl.BlockSpec((1, 512), lambda _: (0, 0)),
                  pl.BlockSpec((512, 512), lambda _: (0, 0)),
                  pl.BlockSpec((1, 512), lambda _: (0, 0))],
        out_specs=pl.BlockSpec((B, 512), lambda _: (0, 0)),
        out_shape=jax.ShapeDtypeStruct((B, 512), jnp.float32),
    )(glob.reshape(B, 1024), W_m2_1, row2(b_m2_1), W_m2_2, row2(b_m2_2))

    return c[:, :, None]


try:
    from jax import shard_map as _shard_map
except ImportError:
    from jax.experimental.shard_map import shard_map as _shard_map

_SM_KW = {}
_sm_params = inspect.signature(_shard_map).parameters
if "check_rep" in _sm_params:
    _SM_KW["check_rep"] = False
elif "check_vma" in _sm_params:
    _SM_KW["check_vma"] = False


@jax.jit
def kernel(pts, W_m1_1, b_m1_1, W_m1_2, b_m1_2, W_m1_3, b_m1_3,
           W_g1_1, b_g1_1, W_g1_2, b_g1_2, W_g2_1, b_g2_1, W_g2_2, b_g2_2,
           W_m2_1, b_m2_1, W_m2_2, b_m2_2):
    args = (pts, W_m1_1, b_m1_1, W_m1_2, b_m1_2, W_m1_3, b_m1_3,
            W_g1_1, b_g1_1, W_g1_2, b_g1_2, W_g2_1, b_g2_1, W_g2_2, b_g2_2,
            W_m2_1, b_m2_1, W_m2_2, b_m2_2)
    devs = jax.devices()
    if len(devs) >= 2 and pts.shape[0] % 2 == 0:
        mesh = jax.sharding.Mesh(np.asarray(devs[:2]), ("d",))
        P = jax.sharding.PartitionSpec
        in_specs = (P("d"),) + (P(),) * 18
        f = _shard_map(_pipeline, mesh=mesh, in_specs=in_specs,
                       out_specs=P("d"), **_SM_KW)
        return f(*args)
    return _pipeline(*args)
